# baseline (speedup 1.0000x reference)
import jax
import jax.numpy as jnp
from jax.experimental import pallas as pl

_NUM_RAYS = 1024
_NUM_LIGHTS = 32
_BATCH = 4096
_RB = _NUM_RAYS * 3
_NB = 256


def _light_kernel(idx_ref, pt_ref, ld_ref, li_ref):
    idxv = idx_ref[...]
    lanes = jax.lax.broadcasted_iota(jnp.int32, (_NB, _NUM_LIGHTS), 1)
    oh = (lanes == idxv).astype(jnp.float32)
    pt = pt_ref[...]
    x = jnp.sum(oh * pt[0:1, :], axis=1, keepdims=True)
    y = jnp.sum(oh * pt[1:2, :], axis=1, keepdims=True)
    z = -jnp.abs(jnp.sum(oh * pt[2:3, :], axis=1, keepdims=True))
    inten = jnp.abs(jnp.sum(oh * pt[3:4, :], axis=1, keepdims=True))
    n = jnp.sqrt(x * x + y * y + z * z)
    inv = 1.0 / jnp.maximum(n, 1e-12)
    xn, yn, zn = x * inv, y * inv, z * inv
    c = jax.lax.broadcasted_iota(jnp.int32, (_NB, _RB), 1) % 3
    ld_ref[...] = jnp.where(c == 0, xn, jnp.where(c == 1, yn, zn))
    li_ref[...] = jnp.broadcast_to(inten, (_NB, _RB))


def kernel(idx, light_direction_xy, light_direction_z, light_intensity):
    params_t = jnp.concatenate(
        [light_direction_xy, light_direction_z, light_intensity], axis=1
    ).T
    idx2 = idx.reshape(_BATCH, 1)

    grid = _BATCH // _NB
    out_ld, out_li = pl.pallas_call(
        _light_kernel,
        grid=(grid,),
        in_specs=[
            pl.BlockSpec((_NB, 1), lambda i: (i, 0)),
            pl.BlockSpec((4, _NUM_LIGHTS), lambda i: (0, 0)),
        ],
        out_specs=[
            pl.BlockSpec((_NB, _RB), lambda i: (i, 0)),
            pl.BlockSpec((_NB, _RB), lambda i: (i, 0)),
        ],
        out_shape=[
            jax.ShapeDtypeStruct((_BATCH, _RB), jnp.float32),
            jax.ShapeDtypeStruct((_BATCH, _RB), jnp.float32),
        ],
    )(idx2, params_t)
    return (out_ld.reshape(-1, 3), out_li.reshape(-1, 3))
